# baseline (speedup 1.0000x reference)
import functools

import jax
import jax.numpy as jnp
import numpy as np
from jax import lax
from jax.experimental import pallas as pl
from jax.experimental.pallas import tpu as pltpu
from jax.experimental.pallas import tpu_sc as plsc

SILU_NORM = 1.679177
NUM_NEIGHBORS = 32.0

NC = 2
NS = 16
NW = NC * NS

LANES = 16


def _silu(x):
    return x / (1.0 + jnp.exp(-x))


def _edge_weights_body(es_ref, attr_ref, w1_ref, w2_ref, o_ref):
    x = es_ref[...]
    scale = attr_ref[...] * (1.0 / np.sqrt(NUM_NEIGHBORS))
    inv1 = 1.0 / np.sqrt(w1_ref.shape[0])
    inv2 = 1.0 / np.sqrt(w2_ref.shape[0])
    h = lax.dot(x, w1_ref[...] * inv1,
                precision=lax.Precision.HIGHEST,
                preferred_element_type=jnp.float32)
    h = SILU_NORM * _silu(h)
    w2 = w2_ref[...] * inv2
    h_hi = h.astype(jnp.bfloat16)
    w2_hi = w2.astype(jnp.bfloat16)
    h_lo = (h - h_hi.astype(jnp.float32)).astype(jnp.bfloat16)
    w2_lo = (w2 - w2_hi.astype(jnp.float32)).astype(jnp.bfloat16)
    acc = lax.dot(h_hi, w2_hi, preferred_element_type=jnp.float32)
    acc += lax.dot(h_lo, w2_hi, preferred_element_type=jnp.float32)
    acc += lax.dot(h_hi, w2_lo, preferred_element_type=jnp.float32)
    o_ref[...] = (acc * scale).astype(jnp.bfloat16)


def _edge_weights(edge_scalars, edge_attr, fc_w1, fc_w2):
    E, NB = edge_scalars.shape
    D = fc_w2.shape[1]
    BE = 6400
    assert E % BE == 0
    grid = (E // BE,)
    full = lambda shape: pl.BlockSpec(shape, lambda i: (0, 0))
    return pl.pallas_call(
        _edge_weights_body,
        grid=grid,
        in_specs=[
            pl.BlockSpec((BE, NB), lambda i: (i, 0)),
            pl.BlockSpec((BE, 1), lambda i: (i, 0)),
            full(fc_w1.shape), full(fc_w2.shape),
        ],
        out_specs=pl.BlockSpec((BE, D), lambda i: (i, 0)),
        out_shape=jax.ShapeDtypeStruct((E, D), jnp.bfloat16),
    )(edge_scalars, edge_attr, fc_w1, fc_w2)


CH = 64


def _pack_worker_chunks(E):
    NCH = E // CH
    NF = NCH // NW
    rem = NCH % NW
    NF1 = NF + (1 if rem else 0)
    return NCH, NF, rem, NF1


def _make_sc_conv(NPAD, E, D):
    NCH, NF, rem, NF1 = _pack_worker_chunks(E)
    RPS = NPAD // NS
    SR = CH
    assert RPS % SR == 0
    NSTAGE = RPS // SR

    mesh = plsc.VectorSubcoreMesh(core_axis_name="c", subcore_axis_name="s")

    def body(node_hbm, srcp_hbm, dstp_hbm, w_hbm, out_hbm,
             acc, src_all, dst32, nodes, wts, gsem, wsem, ssem, dsem):
        cid = lax.axis_index("c")
        sid = lax.axis_index("s")
        wid = sid * NC + cid
        trips = NF + jnp.where(wid < rem, 1, 0)
        start_row = (wid * NF + jnp.minimum(wid, rem)) * CH

        stage = nodes.at[0]

        def zrow(i, _):
            for c in range(D // LANES):
                nodes[0, i, pl.ds(c * LANES, LANES)] = jnp.zeros(
                    (LANES,), jnp.float32)
            return 0
        lax.fori_loop(0, SR, zrow, 0)
        for k in range(NSTAGE):
            pltpu.sync_copy(stage, acc.at[pl.ds(sid * RPS + k * SR, SR)])

        pltpu.sync_copy(srcp_hbm.at[pl.ds(start_row, NF1 * CH)], src_all)
        plsc.subcore_barrier()

        def drain(sem, ref):
            pltpu.make_async_copy(w_hbm.at[pl.ds(0, ref.shape[0])],
                                  ref, sem).wait()

        def draini(sem):
            pltpu.make_async_copy(dstp_hbm.at[pl.ds(0, CH)],
                                  dst32.at[0], sem).wait()

        def step(i, _):
            b = lax.rem(i, 2)
            nb = 1 - b

            @pl.when(i < trips)
            def _():
                @pl.when(i >= 2)
                def _():
                    drain(ssem.at[b], nodes.at[b])
                pltpu.async_copy(node_hbm.at[src_all.at[pl.ds(i * CH, CH)]],
                                 nodes.at[b], gsem.at[b])
                pltpu.async_copy(w_hbm.at[pl.ds(start_row + i * CH, CH)],
                                 wts.at[b], wsem.at[b])
                pltpu.async_copy(
                    dstp_hbm.at[pl.ds(start_row + i * CH, CH)],
                    dst32.at[b], dsem.at[b])

            @pl.when(i >= 1)
            def _():
                drain(gsem.at[nb], nodes.at[nb])
                drain(wsem.at[nb], wts.at[nb])
                draini(dsem.at[nb])

                def mul_slot(s):
                    @plsc.parallel_loop(0, CH, step=1, unroll=4)
                    def _(r):
                        for c in range(D // (2 * LANES)):
                            wv = wts[s, r, pl.ds(c * 2 * LANES, 2 * LANES)]
                            wa, wb = plsc.unpack(
                                wv, format=plsc.PackFormat.INTERLEAVED,
                                preferred_element_type=jnp.float32)
                            sl0 = pl.ds(c * 2 * LANES, LANES)
                            sl1 = pl.ds(c * 2 * LANES + LANES, LANES)
                            nodes[s, r, sl0] = nodes[s, r, sl0] * wa
                            nodes[s, r, sl1] = nodes[s, r, sl1] * wb

                @pl.when(nb == 0)
                def _():
                    mul_slot(0)

                @pl.when(nb == 1)
                def _():
                    mul_slot(1)
                pltpu.async_copy(nodes.at[nb], acc.at[dst32.at[nb]],
                                 ssem.at[nb], add=True)
            return 0
        lax.fori_loop(0, trips + 1, step, 0)

        drain(ssem.at[0], nodes.at[0])
        drain(ssem.at[1], nodes.at[1])

        plsc.subcore_barrier()
        for k in range(NSTAGE):
            rows = pl.ds(sid * RPS + k * SR, SR)
            pltpu.sync_copy(acc.at[rows], stage)
            pltpu.sync_copy(stage, out_hbm.at[cid, rows])

    return pl.kernel(
        body,
        out_type=jax.ShapeDtypeStruct((NC, NPAD, D), jnp.float32),
        mesh=mesh,
        compiler_params=pltpu.CompilerParams(use_tc_tiling_on_sc=False,
                                             needs_layout_passes=False),
        scratch_types=[
            pltpu.VMEM_SHARED((NPAD, D), jnp.float32),
            pltpu.VMEM((NF1 * CH,), jnp.int32),
            pltpu.VMEM((2, CH), jnp.int32),
            pltpu.VMEM((2, CH, D), jnp.float32),
            pltpu.VMEM((2, CH, D), jnp.bfloat16),
            pltpu.SemaphoreType.DMA((2,)),
            pltpu.SemaphoreType.DMA((2,)),
            pltpu.SemaphoreType.DMA((2,)),
            pltpu.SemaphoreType.DMA((2,)),
        ],
    )


def _combine(partials, apply_silu):
    _, N, D = partials.shape
    BR = 2048
    assert N % BR == 0

    def body(p0_ref, p1_ref, o_ref):
        s = p0_ref[0] + p1_ref[0]
        if apply_silu:
            s = SILU_NORM * _silu(s)
        o_ref[...] = s

    return pl.pallas_call(
        body,
        grid=(N // BR,),
        in_specs=[
            pl.BlockSpec((1, BR, D), lambda i: (0, i, 0)),
            pl.BlockSpec((1, BR, D), lambda i: (1, i, 0)),
        ],
        out_specs=pl.BlockSpec((BR, D), lambda i: (i, 0)),
        out_shape=jax.ShapeDtypeStruct((N, D), jnp.float32),
    )(partials, partials)


def kernel(node_input, edge_src, edge_dst, edge_attr, edge_scalars,
           fc0_w1, fc0_w2, fc1_w1, fc1_w2):
    N, D = node_input.shape
    E = edge_src.shape[0]
    NPAD = -(-N // (NS * 128)) * (NS * 128)

    perm = np.arange(D).reshape(-1, 2, 16).transpose(0, 2, 1).reshape(-1)
    w0 = _edge_weights(edge_scalars, edge_attr, fc0_w1, fc0_w2[:, perm])
    w1 = _edge_weights(edge_scalars, edge_attr, fc1_w1, fc1_w2[:, perm])

    sc_conv = _make_sc_conv(NPAD, E, D)

    src_p = jnp.pad(edge_src.astype(jnp.int32), (0, CH))
    dst_p = jnp.pad(edge_dst.astype(jnp.int32), (0, CH))

    p0 = sc_conv(node_input, src_p, dst_p, w0)
    h = _combine(p0, apply_silu=True)
    p1 = sc_conv(h, src_p, dst_p, w1)
    return _combine(p1, apply_silu=False)[:N]


# speedup vs baseline: 1.6899x; 1.6899x over previous
import functools

import jax
import jax.numpy as jnp
import numpy as np
from jax import lax
from jax.experimental import pallas as pl
from jax.experimental.pallas import tpu as pltpu
from jax.experimental.pallas import tpu_sc as plsc

SILU_NORM = 1.679177
NUM_NEIGHBORS = 32.0

NC = 2
NS = 16
NW = NC * NS

LANES = 16


def _silu(x):
    return x / (1.0 + jnp.exp(-x))


def _edge_weights_body(es_ref, attr_ref, w1_ref, w2_ref, o_ref):
    x = es_ref[...]
    scale = attr_ref[...] * (1.0 / np.sqrt(NUM_NEIGHBORS))
    inv1 = 1.0 / np.sqrt(w1_ref.shape[0])
    inv2 = 1.0 / np.sqrt(w2_ref.shape[0])
    def dot3(a, b):
        a_hi = a.astype(jnp.bfloat16)
        b_hi = b.astype(jnp.bfloat16)
        a_lo = (a - a_hi.astype(jnp.float32)).astype(jnp.bfloat16)
        b_lo = (b - b_hi.astype(jnp.float32)).astype(jnp.bfloat16)
        out = lax.dot(a_hi, b_hi, preferred_element_type=jnp.float32)
        out += lax.dot(a_lo, b_hi, preferred_element_type=jnp.float32)
        out += lax.dot(a_hi, b_lo, preferred_element_type=jnp.float32)
        return out

    h = dot3(x, w1_ref[...] * inv1)
    h = SILU_NORM * _silu(h)
    o_ref[...] = dot3(h, w2_ref[...] * inv2) * scale


def _edge_weights(edge_scalars, edge_attr, fc_w1, fc_w2):
    E, NB = edge_scalars.shape
    D = fc_w2.shape[1]
    BE = 6400
    assert E % BE == 0
    grid = (E // BE,)
    full = lambda shape: pl.BlockSpec(shape, lambda i: (0, 0))
    return pl.pallas_call(
        _edge_weights_body,
        grid=grid,
        in_specs=[
            pl.BlockSpec((BE, NB), lambda i: (i, 0)),
            pl.BlockSpec((BE, 1), lambda i: (i, 0)),
            full(fc_w1.shape), full(fc_w2.shape),
        ],
        out_specs=pl.BlockSpec((BE, D), lambda i: (i, 0)),
        out_shape=jax.ShapeDtypeStruct((E, D), jnp.float32),
    )(edge_scalars, edge_attr, fc_w1, fc_w2)


CH = 64


def _pack_worker_chunks(E):
    NCH = E // CH
    NF = NCH // NW
    rem = NCH % NW
    NF1 = NF + (1 if rem else 0)
    return NCH, NF, rem, NF1


def _make_sc_conv(NPAD, E, D):
    NCH, NF, rem, NF1 = _pack_worker_chunks(E)
    RPS = NPAD // NS
    SR = CH
    assert RPS % SR == 0
    NSTAGE = RPS // SR

    mesh = plsc.VectorSubcoreMesh(core_axis_name="c", subcore_axis_name="s")

    def body(node_hbm, srcp_hbm, dstp_hbm, w_hbm, out_hbm,
             acc, src_all, dst32, nodes, wts, gsem, wsem, ssem, dsem):
        cid = lax.axis_index("c")
        sid = lax.axis_index("s")
        wid = sid * NC + cid
        trips = NF + jnp.where(wid < rem, 1, 0)
        start_row = (wid * NF + jnp.minimum(wid, rem)) * CH

        stage = nodes.at[0]

        def zrow(i, _):
            for c in range(D // LANES):
                nodes[0, i, pl.ds(c * LANES, LANES)] = jnp.zeros(
                    (LANES,), jnp.float32)
            return 0
        lax.fori_loop(0, SR, zrow, 0)
        for k in range(NSTAGE):
            pltpu.sync_copy(stage, acc.at[pl.ds(sid * RPS + k * SR, SR)])

        pltpu.sync_copy(srcp_hbm.at[pl.ds(start_row, NF1 * CH)], src_all)
        plsc.subcore_barrier()

        def drain(sem, ref):
            pltpu.make_async_copy(w_hbm.at[pl.ds(0, ref.shape[0])],
                                  ref, sem).wait()

        def draini(sem):
            pltpu.make_async_copy(dstp_hbm.at[pl.ds(0, CH)],
                                  dst32.at[0], sem).wait()

        def step(i, _):
            b = lax.rem(i, 2)
            nb = 1 - b

            @pl.when(i < trips)
            def _():
                @pl.when(i >= 2)
                def _():
                    drain(ssem.at[b], nodes.at[b])
                pltpu.async_copy(node_hbm.at[src_all.at[pl.ds(i * CH, CH)]],
                                 nodes.at[b], gsem.at[b])
                pltpu.async_copy(w_hbm.at[pl.ds(start_row + i * CH, CH)],
                                 wts.at[b], wsem.at[b])
                pltpu.async_copy(
                    dstp_hbm.at[pl.ds(start_row + i * CH, CH)],
                    dst32.at[b], dsem.at[b])

            @pl.when(i >= 1)
            def _():
                drain(gsem.at[nb], nodes.at[nb])
                drain(wsem.at[nb], wts.at[nb])
                draini(dsem.at[nb])

                def mul_slot(s):
                    @plsc.parallel_loop(0, CH, step=1, unroll=4)
                    def _(r):
                        for c in range(D // LANES):
                            sl = pl.ds(c * LANES, LANES)
                            nodes[s, r, sl] = nodes[s, r, sl] * wts[s, r, sl]

                @pl.when(nb == 0)
                def _():
                    mul_slot(0)

                @pl.when(nb == 1)
                def _():
                    mul_slot(1)
                pltpu.async_copy(nodes.at[nb], acc.at[dst32.at[nb]],
                                 ssem.at[nb], add=True)
            return 0
        lax.fori_loop(0, trips + 1, step, 0)

        drain(ssem.at[0], nodes.at[0])
        drain(ssem.at[1], nodes.at[1])

        plsc.subcore_barrier()
        for k in range(NSTAGE):
            rows = pl.ds(sid * RPS + k * SR, SR)
            pltpu.sync_copy(acc.at[rows], stage)
            pltpu.sync_copy(stage, out_hbm.at[cid, rows])

    return pl.kernel(
        body,
        out_type=jax.ShapeDtypeStruct((NC, NPAD, D), jnp.float32),
        mesh=mesh,
        compiler_params=pltpu.CompilerParams(use_tc_tiling_on_sc=False),
        scratch_types=[
            pltpu.VMEM_SHARED((NPAD, D), jnp.float32),
            pltpu.VMEM((NF1 * CH,), jnp.int32),
            pltpu.VMEM((2, CH), jnp.int32),
            pltpu.VMEM((2, CH, D), jnp.float32),
            pltpu.VMEM((2, CH, D), jnp.float32),
            pltpu.SemaphoreType.DMA((2,)),
            pltpu.SemaphoreType.DMA((2,)),
            pltpu.SemaphoreType.DMA((2,)),
            pltpu.SemaphoreType.DMA((2,)),
        ],
    )


def _combine(partials, apply_silu):
    _, N, D = partials.shape
    BR = 2048
    assert N % BR == 0

    def body(p0_ref, p1_ref, o_ref):
        s = p0_ref[0] + p1_ref[0]
        if apply_silu:
            s = SILU_NORM * _silu(s)
        o_ref[...] = s

    return pl.pallas_call(
        body,
        grid=(N // BR,),
        in_specs=[
            pl.BlockSpec((1, BR, D), lambda i: (0, i, 0)),
            pl.BlockSpec((1, BR, D), lambda i: (1, i, 0)),
        ],
        out_specs=pl.BlockSpec((BR, D), lambda i: (i, 0)),
        out_shape=jax.ShapeDtypeStruct((N, D), jnp.float32),
    )(partials, partials)


def kernel(node_input, edge_src, edge_dst, edge_attr, edge_scalars,
           fc0_w1, fc0_w2, fc1_w1, fc1_w2):
    N, D = node_input.shape
    E = edge_src.shape[0]
    NPAD = -(-N // (NS * 128)) * (NS * 128)

    w0 = _edge_weights(edge_scalars, edge_attr, fc0_w1, fc0_w2)
    w1 = _edge_weights(edge_scalars, edge_attr, fc1_w1, fc1_w2)

    sc_conv = _make_sc_conv(NPAD, E, D)

    src_p = jnp.pad(edge_src.astype(jnp.int32), (0, CH))
    dst_p = jnp.pad(edge_dst.astype(jnp.int32), (0, CH))

    p0 = sc_conv(node_input, src_p, dst_p, w0)
    h = _combine(p0, apply_silu=True)
    p1 = sc_conv(h, src_p, dst_p, w1)
    return _combine(p1, apply_silu=False)[:N]


# speedup vs baseline: 1.8646x; 1.1033x over previous
import functools

import jax
import jax.numpy as jnp
import numpy as np
from jax import lax
from jax.experimental import pallas as pl
from jax.experimental.pallas import tpu as pltpu
from jax.experimental.pallas import tpu_sc as plsc

SILU_NORM = 1.679177
NUM_NEIGHBORS = 32.0

NC = 2
NS = 16
NW = NC * NS

LANES = 16


def _silu(x):
    return x / (1.0 + jnp.exp(-x))


def _edge_weights_body(es_ref, attr_ref, w1_ref, w2_ref, o_ref):
    x = es_ref[...]
    scale = attr_ref[...] * (1.0 / np.sqrt(NUM_NEIGHBORS))
    inv1 = 1.0 / np.sqrt(w1_ref.shape[0])
    inv2 = 1.0 / np.sqrt(w2_ref.shape[0])
    h = lax.dot(x, w1_ref[...] * inv1, preferred_element_type=jnp.float32)
    h = SILU_NORM * _silu(h)
    o_ref[...] = lax.dot(h, w2_ref[...] * inv2,
                         preferred_element_type=jnp.float32) * scale


def _edge_weights(edge_scalars, edge_attr, fc_w1, fc_w2):
    E, NB = edge_scalars.shape
    D = fc_w2.shape[1]
    BE = 6400
    assert E % BE == 0
    grid = (E // BE,)
    full = lambda shape: pl.BlockSpec(shape, lambda i: (0, 0))
    return pl.pallas_call(
        _edge_weights_body,
        grid=grid,
        in_specs=[
            pl.BlockSpec((BE, NB), lambda i: (i, 0)),
            pl.BlockSpec((BE, 1), lambda i: (i, 0)),
            full(fc_w1.shape), full(fc_w2.shape),
        ],
        out_specs=pl.BlockSpec((BE, D), lambda i: (i, 0)),
        out_shape=jax.ShapeDtypeStruct((E, D), jnp.float32),
    )(edge_scalars, edge_attr, fc_w1, fc_w2)


CH = 64


def _pack_worker_chunks(E):
    NCH = E // CH
    NF = NCH // NW
    rem = NCH % NW
    NF1 = NF + (1 if rem else 0)
    return NCH, NF, rem, NF1


def _make_sc_conv(NPAD, E, D):
    NCH, NF, rem, NF1 = _pack_worker_chunks(E)
    RPS = NPAD // NS
    SR = CH
    assert RPS % SR == 0
    NSTAGE = RPS // SR

    mesh = plsc.VectorSubcoreMesh(core_axis_name="c", subcore_axis_name="s")

    def body(node_hbm, srcp_hbm, dstp_hbm, w_hbm, out_hbm,
             acc, src_all, dst32, nodes, wts, gsem, wsem, ssem, dsem):
        cid = lax.axis_index("c")
        sid = lax.axis_index("s")
        wid = sid * NC + cid
        trips = NF + jnp.where(wid < rem, 1, 0)
        start_row = (wid * NF + jnp.minimum(wid, rem)) * CH

        stage = nodes.at[0]

        def zrow(i, _):
            for c in range(D // LANES):
                nodes[0, i, pl.ds(c * LANES, LANES)] = jnp.zeros(
                    (LANES,), jnp.float32)
            return 0
        lax.fori_loop(0, SR, zrow, 0)
        for k in range(NSTAGE):
            pltpu.sync_copy(stage, acc.at[pl.ds(sid * RPS + k * SR, SR)])

        pltpu.sync_copy(srcp_hbm.at[pl.ds(start_row, NF1 * CH)], src_all)
        plsc.subcore_barrier()

        def drain(sem, ref):
            pltpu.make_async_copy(w_hbm.at[pl.ds(0, ref.shape[0])],
                                  ref, sem).wait()

        def draini(sem):
            pltpu.make_async_copy(dstp_hbm.at[pl.ds(0, CH)],
                                  dst32.at[0], sem).wait()

        def step(i, _):
            b = lax.rem(i, 2)
            nb = 1 - b

            @pl.when(i < trips)
            def _():
                @pl.when(i >= 2)
                def _():
                    drain(ssem.at[b], nodes.at[b])
                pltpu.async_copy(node_hbm.at[src_all.at[pl.ds(i * CH, CH)]],
                                 nodes.at[b], gsem.at[b])
                pltpu.async_copy(w_hbm.at[pl.ds(start_row + i * CH, CH)],
                                 wts.at[b], wsem.at[b])
                pltpu.async_copy(
                    dstp_hbm.at[pl.ds(start_row + i * CH, CH)],
                    dst32.at[b], dsem.at[b])

            @pl.when(i >= 1)
            def _():
                drain(gsem.at[nb], nodes.at[nb])
                drain(wsem.at[nb], wts.at[nb])
                draini(dsem.at[nb])

                def mul_slot(s):
                    @plsc.parallel_loop(0, CH, step=1, unroll=4)
                    def _(r):
                        for c in range(D // LANES):
                            sl = pl.ds(c * LANES, LANES)
                            nodes[s, r, sl] = nodes[s, r, sl] * wts[s, r, sl]

                @pl.when(nb == 0)
                def _():
                    mul_slot(0)

                @pl.when(nb == 1)
                def _():
                    mul_slot(1)
                pltpu.async_copy(nodes.at[nb], acc.at[dst32.at[nb]],
                                 ssem.at[nb], add=True)
            return 0
        lax.fori_loop(0, trips + 1, step, 0)

        drain(ssem.at[0], nodes.at[0])
        drain(ssem.at[1], nodes.at[1])

        plsc.subcore_barrier()
        for k in range(NSTAGE):
            rows = pl.ds(sid * RPS + k * SR, SR)
            pltpu.sync_copy(acc.at[rows], stage)
            pltpu.sync_copy(stage, out_hbm.at[cid, rows])

    return pl.kernel(
        body,
        out_type=jax.ShapeDtypeStruct((NC, NPAD, D), jnp.float32),
        mesh=mesh,
        compiler_params=pltpu.CompilerParams(use_tc_tiling_on_sc=False),
        scratch_types=[
            pltpu.VMEM_SHARED((NPAD, D), jnp.float32),
            pltpu.VMEM((NF1 * CH,), jnp.int32),
            pltpu.VMEM((2, CH), jnp.int32),
            pltpu.VMEM((2, CH, D), jnp.float32),
            pltpu.VMEM((2, CH, D), jnp.float32),
            pltpu.SemaphoreType.DMA((2,)),
            pltpu.SemaphoreType.DMA((2,)),
            pltpu.SemaphoreType.DMA((2,)),
            pltpu.SemaphoreType.DMA((2,)),
        ],
    )


def _combine(partials, apply_silu):
    _, N, D = partials.shape
    BR = 2048
    assert N % BR == 0

    def body(p0_ref, p1_ref, o_ref):
        s = p0_ref[0] + p1_ref[0]
        if apply_silu:
            s = SILU_NORM * _silu(s)
        o_ref[...] = s

    return pl.pallas_call(
        body,
        grid=(N // BR,),
        in_specs=[
            pl.BlockSpec((1, BR, D), lambda i: (0, i, 0)),
            pl.BlockSpec((1, BR, D), lambda i: (1, i, 0)),
        ],
        out_specs=pl.BlockSpec((BR, D), lambda i: (i, 0)),
        out_shape=jax.ShapeDtypeStruct((N, D), jnp.float32),
    )(partials, partials)


def kernel(node_input, edge_src, edge_dst, edge_attr, edge_scalars,
           fc0_w1, fc0_w2, fc1_w1, fc1_w2):
    N, D = node_input.shape
    E = edge_src.shape[0]
    NPAD = -(-N // (NS * 128)) * (NS * 128)

    w0 = _edge_weights(edge_scalars, edge_attr, fc0_w1, fc0_w2)
    w1 = _edge_weights(edge_scalars, edge_attr, fc1_w1, fc1_w2)

    sc_conv = _make_sc_conv(NPAD, E, D)

    src_p = jnp.pad(edge_src.astype(jnp.int32), (0, CH))
    dst_p = jnp.pad(edge_dst.astype(jnp.int32), (0, CH))

    p0 = sc_conv(node_input, src_p, dst_p, w0)
    h = _combine(p0, apply_silu=True)
    p1 = sc_conv(h, src_p, dst_p, w1)
    return _combine(p1, apply_silu=False)[:N]


# speedup vs baseline: 1.8649x; 1.0002x over previous
import functools

import jax
import jax.numpy as jnp
import numpy as np
from jax import lax
from jax.experimental import pallas as pl
from jax.experimental.pallas import tpu as pltpu
from jax.experimental.pallas import tpu_sc as plsc

SILU_NORM = 1.679177
NUM_NEIGHBORS = 32.0

NC = 2
NS = 16
NW = NC * NS

LANES = 16


def _silu(x):
    return x / (1.0 + jnp.exp(-x))


def _edge_weights_body(es_ref, attr_ref, w1_ref, w2_ref, o_ref):
    x = es_ref[...]
    scale = attr_ref[...] * (1.0 / np.sqrt(NUM_NEIGHBORS))
    inv1 = 1.0 / np.sqrt(w1_ref.shape[0])
    inv2 = 1.0 / np.sqrt(w2_ref.shape[0])
    h = lax.dot(x, w1_ref[...] * inv1, preferred_element_type=jnp.float32)
    h = SILU_NORM * _silu(h)
    o_ref[...] = lax.dot(h, w2_ref[...] * inv2,
                         preferred_element_type=jnp.float32) * scale


def _edge_weights(edge_scalars, edge_attr, fc_w1, fc_w2):
    E, NB = edge_scalars.shape
    D = fc_w2.shape[1]
    BE = 6400
    assert E % BE == 0
    grid = (E // BE,)
    full = lambda shape: pl.BlockSpec(shape, lambda i: (0, 0))
    return pl.pallas_call(
        _edge_weights_body,
        grid=grid,
        in_specs=[
            pl.BlockSpec((BE, NB), lambda i: (i, 0)),
            pl.BlockSpec((BE, 1), lambda i: (i, 0)),
            full(fc_w1.shape), full(fc_w2.shape),
        ],
        out_specs=pl.BlockSpec((BE, D), lambda i: (i, 0)),
        out_shape=jax.ShapeDtypeStruct((E, D), jnp.float32),
    )(edge_scalars, edge_attr, fc_w1, fc_w2)


CH = 64


def _pack_worker_chunks(E):
    NCH = E // CH
    NF = NCH // NW
    rem = NCH % NW
    NF1 = NF + (1 if rem else 0)
    return NCH, NF, rem, NF1


def _make_sc_conv(NPAD, E, D):
    NCH, NF, rem, NF1 = _pack_worker_chunks(E)
    RPS = NPAD // NS
    SR = CH
    assert RPS % SR == 0
    NSTAGE = RPS // SR

    mesh = plsc.VectorSubcoreMesh(core_axis_name="c", subcore_axis_name="s")

    def body(node_hbm, srcp_hbm, dstp_hbm, w_hbm, out_hbm,
             acc, src_all, dst32, nodes, wts, gsem, wsem, ssem, dsem):
        cid = lax.axis_index("c")
        sid = lax.axis_index("s")
        wid = sid * NC + cid
        trips = NF + jnp.where(wid < rem, 1, 0)
        start_row = (wid * NF + jnp.minimum(wid, rem)) * CH

        stage = nodes.at[0]

        def zrow(i, _):
            for c in range(D // LANES):
                nodes[0, i, pl.ds(c * LANES, LANES)] = jnp.zeros(
                    (LANES,), jnp.float32)
            return 0
        lax.fori_loop(0, SR, zrow, 0)
        for k in range(NSTAGE):
            pltpu.sync_copy(stage, acc.at[pl.ds(sid * RPS + k * SR, SR)])

        pltpu.sync_copy(srcp_hbm.at[pl.ds(start_row, NF1 * CH)], src_all)
        plsc.subcore_barrier()

        def drain(sem, ref):
            pltpu.make_async_copy(w_hbm.at[pl.ds(0, ref.shape[0])],
                                  ref, sem).wait()

        def draini(sem):
            pltpu.make_async_copy(dstp_hbm.at[pl.ds(0, CH)],
                                  dst32.at[0], sem).wait()

        def step(i, _):
            b = lax.rem(i, 2)
            nb = 1 - b

            @pl.when(i < trips)
            def _():
                @pl.when(i >= 2)
                def _():
                    drain(ssem.at[b], nodes.at[b])
                pltpu.async_copy(node_hbm.at[src_all.at[pl.ds(i * CH, CH)]],
                                 nodes.at[b], gsem.at[b])
                pltpu.async_copy(w_hbm.at[pl.ds(start_row + i * CH, CH)],
                                 wts.at[b], wsem.at[b])
                pltpu.async_copy(
                    dstp_hbm.at[pl.ds(start_row + i * CH, CH)],
                    dst32.at[b], dsem.at[b])

            @pl.when(i >= 1)
            def _():
                drain(gsem.at[nb], nodes.at[nb])
                drain(wsem.at[nb], wts.at[nb])
                draini(dsem.at[nb])

                def mul_slot(s):
                    @plsc.parallel_loop(0, CH, step=1, unroll=4)
                    def _(r):
                        for c in range(D // LANES):
                            sl = pl.ds(c * LANES, LANES)
                            nodes[s, r, sl] = nodes[s, r, sl] * wts[s, r, sl]

                @pl.when(nb == 0)
                def _():
                    mul_slot(0)

                @pl.when(nb == 1)
                def _():
                    mul_slot(1)
                pltpu.async_copy(nodes.at[nb], acc.at[dst32.at[nb]],
                                 ssem.at[nb], add=True)
            return 0
        lax.fori_loop(0, trips + 1, step, 0)

        drain(ssem.at[0], nodes.at[0])
        drain(ssem.at[1], nodes.at[1])

        plsc.subcore_barrier()
        for k in range(NSTAGE):
            rows = pl.ds(sid * RPS + k * SR, SR)
            pltpu.sync_copy(acc.at[rows], stage)
            pltpu.sync_copy(stage, out_hbm.at[cid, rows])

    return pl.kernel(
        body,
        out_type=jax.ShapeDtypeStruct((NC, NPAD, D), jnp.float32),
        mesh=mesh,
        compiler_params=pltpu.CompilerParams(use_tc_tiling_on_sc=False),
        scratch_types=[
            pltpu.VMEM_SHARED((NPAD, D), jnp.float32),
            pltpu.VMEM((NF1 * CH,), jnp.int32),
            pltpu.VMEM((2, CH), jnp.int32),
            pltpu.VMEM((2, CH, D), jnp.float32),
            pltpu.VMEM((2, CH, D), jnp.float32),
            pltpu.SemaphoreType.DMA((2,)),
            pltpu.SemaphoreType.DMA((2,)),
            pltpu.SemaphoreType.DMA((2,)),
            pltpu.SemaphoreType.DMA((2,)),
        ],
    )


def _combine(partials, apply_silu):
    _, N, D = partials.shape
    BR = 2048
    assert N % BR == 0

    def body(p0_ref, p1_ref, o_ref):
        s = p0_ref[0] + p1_ref[0]
        if apply_silu:
            s = SILU_NORM * _silu(s)
        o_ref[...] = s

    return pl.pallas_call(
        body,
        grid=(N // BR,),
        in_specs=[
            pl.BlockSpec((1, BR, D), lambda i: (0, i, 0)),
            pl.BlockSpec((1, BR, D), lambda i: (1, i, 0)),
        ],
        out_specs=pl.BlockSpec((BR, D), lambda i: (i, 0)),
        out_shape=jax.ShapeDtypeStruct((N, D), jnp.float32),
    )(partials, partials)


def kernel(node_input, edge_src, edge_dst, edge_attr, edge_scalars,
           fc0_w1, fc0_w2, fc1_w1, fc1_w2):
    N, D = node_input.shape
    E = edge_src.shape[0]
    NPAD = -(-N // (NS * 128)) * (NS * 128)

    sc_conv = _make_sc_conv(NPAD, E, D)

    src_p = jnp.pad(edge_src.astype(jnp.int32), (0, CH))
    dst_p = jnp.pad(edge_dst.astype(jnp.int32), (0, CH))

    w0 = _edge_weights(edge_scalars, edge_attr, fc0_w1, fc0_w2)
    p0 = sc_conv(node_input, src_p, dst_p, w0)
    w1 = _edge_weights(edge_scalars, edge_attr, fc1_w1, fc1_w2)
    h = _combine(p0, apply_silu=True)
    p1 = sc_conv(h, src_p, dst_p, w1)
    return _combine(p1, apply_silu=False)[:N]
